# baseline (speedup 1.0000x reference)
import jax
import jax.numpy as jnp
from jax.experimental import pallas as pl
from jax.experimental.pallas import tpu as pltpu

Q, K, D = 1024, 16384, 1536
BQ, BK = 256, 2048
NQ, NK = Q // BQ, K // BK
BK2 = 2048
NK2 = K // BK2
_INF = float("inf")
_EPS = 1e-12


def _stage1_body(q_ref, k_ref, mind_ref, sidx_ref, bstep_ref, sstar_ref,
                 k2out_ref, fmin_ref, fstep_ref):
    j = pl.program_id(0)
    q = q_ref[...]
    NH = 4
    H = BK // NH
    ks = [k_ref[pl.ds(h * H, H), :] for h in range(NH)]
    kqs = [jax.lax.dot_general(kh, q, (((1,), (1,)), ((), ())),
                               preferred_element_type=jnp.float32)
           for kh in ks]
    k2s = [jnp.sum(kh * kh, axis=1, keepdims=True) for kh in ks]
    bmins = [jnp.min(k2 - 2.0 * kq, axis=0, keepdims=True)
             for k2, kq in zip(k2s, kqs)]
    bmin = bmins[0]
    for b in bmins[1:]:
        bmin = jnp.minimum(bmin, b)
    k2out_ref[...] = jnp.concatenate(k2s, axis=0)

    @pl.when(j == 0)
    def _():
        fmin_ref[...] = jnp.full((1, Q), _INF, jnp.float32)
        fstep_ref[...] = jnp.zeros((1, Q), jnp.int32)

    old_min = fmin_ref[...]
    old_step = fstep_ref[...]
    take = bmin < old_min
    new_min = jnp.where(take, bmin, old_min)
    new_step = jnp.where(take, j, old_step)
    fmin_ref[...] = new_min
    fstep_ref[...] = new_step

    @pl.when(j == NK - 1)
    def _():
        q2 = jax.lax.dot_general(jnp.ones((1, D), jnp.float32), q * q,
                                 (((1,), (1,)), ((), ())),
                                 precision=jax.lax.Precision.HIGHEST,
                                 preferred_element_type=jnp.float32)
        d2min = q2 + new_min
        mind_ref[...] = jnp.sqrt(jnp.maximum(d2min, _EPS))
        s_val = jnp.max(d2min)
        qio = jax.lax.broadcasted_iota(jnp.int32, (1, Q), 1)
        s_idx = jnp.min(jnp.where(d2min == s_val, qio, Q))
        bstep = jnp.sum(jnp.where(qio == s_idx, new_step, 0))
        sidx_ref[...] = jnp.full((1, 1), s_idx, jnp.int32)
        bstep_ref[...] = jnp.full((1, 1), bstep, jnp.int32)
        sstar_ref[...] = jnp.full((1, 1), jnp.sqrt(jnp.maximum(s_val, _EPS)),
                                  jnp.float32)


def _star_body(bstep_pref, mt_ref, k_ref, k2_ref, star_ref):
    k = k_ref[...]
    mt = mt_ref[...]
    kq = jax.lax.dot_general(k, mt, (((1,), (1,)), ((), ())),
                             preferred_element_type=jnp.float32)
    e = k2_ref[...] - 2.0 * kq
    v = jnp.min(e)
    io = jax.lax.broadcasted_iota(jnp.int32, (BK, 1), 0)
    loc = jnp.min(jnp.where(e == v, io, BK))
    star_ref[...] = jnp.full((1, 1), loc + bstep_pref[0] * BK, jnp.int32)


def _stage2_body(msmt_ref, k_ref, k2_ref, sstar_ref, out_ref, cand_ref):
    j = pl.program_id(0)
    k = k_ref[...]
    msmt = msmt_ref[...]
    kq = jax.lax.dot_general(k, msmt, (((1,), (1,)), ((), ())),
                             preferred_element_type=jnp.float32)
    m2 = jnp.sum(msmt * msmt, axis=1, keepdims=True)
    k2 = k2_ref[...]
    ds = jnp.sqrt(jnp.maximum(k2 - 2.0 * kq[:, 0:1] + m2[0:1, :], _EPS))
    dt = jnp.sqrt(jnp.maximum(k2 - 2.0 * kq[:, 1:2] + m2[1:2, :], _EPS))
    io = jax.lax.broadcasted_iota(jnp.int32, (BK2, 1), 0)

    def top1(dvec):
        v = jnp.min(dvec)
        i1 = jnp.min(jnp.where(dvec == v, io, BK2))
        pay = jnp.sum(jnp.where(io == i1, dt, 0.0))
        return v, pay, i1

    bv1, bd1, i1 = top1(ds)
    ds_b = jnp.where(io == i1, _INF, ds)
    bv2, bd2, i2 = top1(ds_b)
    ds_c = jnp.where(io == i2, _INF, ds_b)
    bv3, bd3, _ = top1(ds_c)

    @pl.when(j == 0)
    def _():
        cand_ref[0] = _INF
        cand_ref[1] = _INF
        cand_ref[2] = _INF
        cand_ref[3] = 0.0
        cand_ref[4] = 0.0
        cand_ref[5] = 0.0

    rv1, rv2, rv3 = cand_ref[0], cand_ref[1], cand_ref[2]
    rd1, rd2, rd3 = cand_ref[3], cand_ref[4], cand_ref[5]

    c1 = bv1 < rv1
    o1v = jnp.where(c1, bv1, rv1)
    o1d = jnp.where(c1, bd1, rd1)
    a2 = bv1 < rv2
    A2v = jnp.where(a2, bv1, rv2)
    A2d = jnp.where(a2, bd1, rd2)
    A3v = jnp.where(a2, jnp.where(bv2 < rv2, bv2, rv2),
                    jnp.where(bv1 < rv3, bv1, rv3))
    A3d = jnp.where(a2, jnp.where(bv2 < rv2, bd2, rd2),
                    jnp.where(bv1 < rv3, bd1, rd3))
    b2c = bv2 < rv1
    B2v = jnp.where(b2c, bv2, rv1)
    B2d = jnp.where(b2c, bd2, rd1)
    B3v = jnp.where(b2c, jnp.where(bv3 < rv1, bv3, rv1),
                    jnp.where(bv2 < rv2, bv2, rv2))
    B3d = jnp.where(b2c, jnp.where(bv3 < rv1, bd3, rd1),
                    jnp.where(bv2 < rv2, bd2, rd2))
    o2v = jnp.where(c1, B2v, A2v)
    o2d = jnp.where(c1, B2d, A2d)
    o3v = jnp.where(c1, B3v, A3v)
    o3d = jnp.where(c1, B3d, A3d)
    cand_ref[0] = o1v
    cand_ref[1] = o2v
    cand_ref[2] = o3v
    cand_ref[3] = o1d
    cand_ref[4] = o2d
    cand_ref[5] = o3d

    @pl.when(j == NK2 - 1)
    def _():
        dc = jnp.sqrt(jnp.float32(D))
        s_star = sstar_ref[...]
        den = jnp.exp(jnp.full((1, 1), o2d) / dc) + \
            jnp.exp(jnp.full((1, 1), o3d) / dc)
        out_ref[...] = (1.0 - jnp.exp(s_star / dc) / den) * s_star


def kernel(queries, keys):
    min_d, s_idx, bstep, s_star, k2_all = pl.pallas_call(
        _stage1_body,
        grid=(NK,),
        in_specs=[pl.BlockSpec((Q, D), lambda j: (0, 0)),
                  pl.BlockSpec((BK, D), lambda j: (j, 0))],
        out_specs=[pl.BlockSpec((1, Q), lambda j: (0, 0)),
                   pl.BlockSpec((1, 1), lambda j: (0, 0)),
                   pl.BlockSpec((1, 1), lambda j: (0, 0)),
                   pl.BlockSpec((1, 1), lambda j: (0, 0)),
                   pl.BlockSpec((BK, 1), lambda j: (j, 0))],
        out_shape=[jax.ShapeDtypeStruct((1, Q), jnp.float32),
                   jax.ShapeDtypeStruct((1, 1), jnp.int32),
                   jax.ShapeDtypeStruct((1, 1), jnp.int32),
                   jax.ShapeDtypeStruct((1, 1), jnp.float32),
                   jax.ShapeDtypeStruct((K, 1), jnp.float32)],
        scratch_shapes=[pltpu.VMEM((1, Q), jnp.float32),
                        pltpu.VMEM((1, Q), jnp.int32)],
        compiler_params=pltpu.CompilerParams(
            dimension_semantics=("arbitrary",)),
    )(queries, keys)

    m_test = jax.lax.dynamic_slice(queries, (s_idx[0, 0], 0), (1, D))
    star_idx = pl.pallas_call(
        _star_body,
        grid_spec=pltpu.PrefetchScalarGridSpec(
            num_scalar_prefetch=1,
            grid=(1,),
            in_specs=[pl.BlockSpec((1, D), lambda i, b: (0, 0)),
                      pl.BlockSpec((BK, D), lambda i, b: (b[0], 0)),
                      pl.BlockSpec((BK, 1), lambda i, b: (b[0], 0))],
            out_specs=pl.BlockSpec((1, 1), lambda i, b: (0, 0)),
        ),
        out_shape=jax.ShapeDtypeStruct((1, 1), jnp.int32),
    )(bstep.reshape((1,)), m_test, keys, k2_all)
    m_star = jax.lax.dynamic_slice(keys, (star_idx[0, 0], 0), (1, D))
    msmt = jnp.concatenate([m_star, m_test], axis=0)
    score = pl.pallas_call(
        _stage2_body,
        grid=(NK2,),
        in_specs=[pl.BlockSpec((2, D), lambda j: (0, 0)),
                  pl.BlockSpec((BK2, D), lambda j: (j, 0)),
                  pl.BlockSpec((BK2, 1), lambda j: (j, 0)),
                  pl.BlockSpec((1, 1), lambda j: (0, 0))],
        out_specs=pl.BlockSpec((1, 1), lambda j: (0, 0)),
        out_shape=jax.ShapeDtypeStruct((1, 1), jnp.float32),
        scratch_shapes=[pltpu.SMEM((8,), jnp.float32)],
        compiler_params=pltpu.CompilerParams(
            dimension_semantics=("arbitrary",)),
    )(msmt, keys, k2_all, s_star)

    return score[0, 0], min_d.reshape(32, 32)


# speedup vs baseline: 1.0093x; 1.0093x over previous
import jax
import jax.numpy as jnp
from jax.experimental import pallas as pl
from jax.experimental.pallas import tpu as pltpu

Q, K, D = 1024, 16384, 1536
BQ, BK = 256, 2048
NQ, NK = Q // BQ, K // BK
BK2 = 2048
NK2 = K // BK2
_INF = float("inf")
_EPS = 1e-12


def _stage1_body(q_ref, k_ref, mind_ref, sidx_ref, bstep_ref, sstar_ref,
                 fmin_ref, fstep_ref):
    j = pl.program_id(0)
    q = q_ref[...]
    NH = 4
    H = BK // NH
    ks = [k_ref[pl.ds(h * H, H), :] for h in range(NH)]
    kqs = [jax.lax.dot_general(kh, q, (((1,), (1,)), ((), ())),
                               preferred_element_type=jnp.float32)
           for kh in ks]
    k2s = [jnp.sum(kh * kh, axis=1, keepdims=True) for kh in ks]
    bmins = [jnp.min(k2 - 2.0 * kq, axis=0, keepdims=True)
             for k2, kq in zip(k2s, kqs)]
    bmin = bmins[0]
    for b in bmins[1:]:
        bmin = jnp.minimum(bmin, b)

    @pl.when(j == 0)
    def _():
        fmin_ref[...] = jnp.full((1, Q), _INF, jnp.float32)
        fstep_ref[...] = jnp.zeros((1, Q), jnp.int32)

    old_min = fmin_ref[...]
    old_step = fstep_ref[...]
    take = bmin < old_min
    new_min = jnp.where(take, bmin, old_min)
    new_step = jnp.where(take, j, old_step)
    fmin_ref[...] = new_min
    fstep_ref[...] = new_step

    @pl.when(j == NK - 1)
    def _():
        q2 = jax.lax.dot_general(jnp.ones((1, D), jnp.float32), q * q,
                                 (((1,), (1,)), ((), ())),
                                 precision=jax.lax.Precision.HIGHEST,
                                 preferred_element_type=jnp.float32)
        d2min = q2 + new_min
        mind_ref[...] = jnp.sqrt(jnp.maximum(d2min, _EPS))
        s_val = jnp.max(d2min)
        qio = jax.lax.broadcasted_iota(jnp.int32, (1, Q), 1)
        s_idx = jnp.min(jnp.where(d2min == s_val, qio, Q))
        bstep = jnp.sum(jnp.where(qio == s_idx, new_step, 0))
        sidx_ref[...] = jnp.full((1, 1), s_idx, jnp.int32)
        bstep_ref[...] = jnp.full((1, 1), bstep, jnp.int32)
        sstar_ref[...] = jnp.full((1, 1), jnp.sqrt(jnp.maximum(s_val, _EPS)),
                                  jnp.float32)


def _star_body(bstep_pref, mt_ref, k_ref, star_ref):
    k = k_ref[...]
    mt = mt_ref[...]
    kq = jax.lax.dot_general(k, mt, (((1,), (1,)), ((), ())),
                             preferred_element_type=jnp.float32)
    k2 = jnp.sum(k * k, axis=1, keepdims=True)
    e = k2 - 2.0 * kq
    v = jnp.min(e)
    io = jax.lax.broadcasted_iota(jnp.int32, (BK, 1), 0)
    loc = jnp.min(jnp.where(e == v, io, BK))
    star_ref[...] = jnp.full((1, 1), loc + bstep_pref[0] * BK, jnp.int32)


def _stage2_body(msmt_ref, k_ref, sstar_ref, out_ref, cand_ref):
    j = pl.program_id(0)
    k = k_ref[...]
    msmt = msmt_ref[...]
    kq = jax.lax.dot_general(k, msmt, (((1,), (1,)), ((), ())),
                             preferred_element_type=jnp.float32)
    m2 = jnp.sum(msmt * msmt, axis=1, keepdims=True)
    k2 = jnp.sum(k * k, axis=1, keepdims=True)
    ds = jnp.sqrt(jnp.maximum(k2 - 2.0 * kq[:, 0:1] + m2[0:1, :], _EPS))
    dt = jnp.sqrt(jnp.maximum(k2 - 2.0 * kq[:, 1:2] + m2[1:2, :], _EPS))
    io = jax.lax.broadcasted_iota(jnp.int32, (BK2, 1), 0)

    def top1(dvec):
        v = jnp.min(dvec)
        i1 = jnp.min(jnp.where(dvec == v, io, BK2))
        pay = jnp.sum(jnp.where(io == i1, dt, 0.0))
        return v, pay, i1

    bv1, bd1, i1 = top1(ds)
    ds_b = jnp.where(io == i1, _INF, ds)
    bv2, bd2, i2 = top1(ds_b)
    ds_c = jnp.where(io == i2, _INF, ds_b)
    bv3, bd3, _ = top1(ds_c)

    @pl.when(j == 0)
    def _():
        cand_ref[0] = _INF
        cand_ref[1] = _INF
        cand_ref[2] = _INF
        cand_ref[3] = 0.0
        cand_ref[4] = 0.0
        cand_ref[5] = 0.0

    rv1, rv2, rv3 = cand_ref[0], cand_ref[1], cand_ref[2]
    rd1, rd2, rd3 = cand_ref[3], cand_ref[4], cand_ref[5]

    c1 = bv1 < rv1
    o1v = jnp.where(c1, bv1, rv1)
    o1d = jnp.where(c1, bd1, rd1)
    a2 = bv1 < rv2
    A2v = jnp.where(a2, bv1, rv2)
    A2d = jnp.where(a2, bd1, rd2)
    A3v = jnp.where(a2, jnp.where(bv2 < rv2, bv2, rv2),
                    jnp.where(bv1 < rv3, bv1, rv3))
    A3d = jnp.where(a2, jnp.where(bv2 < rv2, bd2, rd2),
                    jnp.where(bv1 < rv3, bd1, rd3))
    b2c = bv2 < rv1
    B2v = jnp.where(b2c, bv2, rv1)
    B2d = jnp.where(b2c, bd2, rd1)
    B3v = jnp.where(b2c, jnp.where(bv3 < rv1, bv3, rv1),
                    jnp.where(bv2 < rv2, bv2, rv2))
    B3d = jnp.where(b2c, jnp.where(bv3 < rv1, bd3, rd1),
                    jnp.where(bv2 < rv2, bd2, rd2))
    o2v = jnp.where(c1, B2v, A2v)
    o2d = jnp.where(c1, B2d, A2d)
    o3v = jnp.where(c1, B3v, A3v)
    o3d = jnp.where(c1, B3d, A3d)
    cand_ref[0] = o1v
    cand_ref[1] = o2v
    cand_ref[2] = o3v
    cand_ref[3] = o1d
    cand_ref[4] = o2d
    cand_ref[5] = o3d

    @pl.when(j == NK2 - 1)
    def _():
        dc = jnp.sqrt(jnp.float32(D))
        s_star = sstar_ref[...]
        den = jnp.exp(jnp.full((1, 1), o2d) / dc) + \
            jnp.exp(jnp.full((1, 1), o3d) / dc)
        out_ref[...] = (1.0 - jnp.exp(s_star / dc) / den) * s_star


def kernel(queries, keys):
    min_d, s_idx, bstep, s_star = pl.pallas_call(
        _stage1_body,
        grid=(NK,),
        in_specs=[pl.BlockSpec((Q, D), lambda j: (0, 0)),
                  pl.BlockSpec((BK, D), lambda j: (j, 0))],
        out_specs=[pl.BlockSpec((1, Q), lambda j: (0, 0)),
                   pl.BlockSpec((1, 1), lambda j: (0, 0)),
                   pl.BlockSpec((1, 1), lambda j: (0, 0)),
                   pl.BlockSpec((1, 1), lambda j: (0, 0))],
        out_shape=[jax.ShapeDtypeStruct((1, Q), jnp.float32),
                   jax.ShapeDtypeStruct((1, 1), jnp.int32),
                   jax.ShapeDtypeStruct((1, 1), jnp.int32),
                   jax.ShapeDtypeStruct((1, 1), jnp.float32)],
        scratch_shapes=[pltpu.VMEM((1, Q), jnp.float32),
                        pltpu.VMEM((1, Q), jnp.int32)],
        compiler_params=pltpu.CompilerParams(
            dimension_semantics=("arbitrary",)),
    )(queries, keys)

    m_test = jax.lax.dynamic_slice(queries, (s_idx[0, 0], 0), (1, D))
    star_idx = pl.pallas_call(
        _star_body,
        grid_spec=pltpu.PrefetchScalarGridSpec(
            num_scalar_prefetch=1,
            grid=(1,),
            in_specs=[pl.BlockSpec((1, D), lambda i, b: (0, 0)),
                      pl.BlockSpec((BK, D), lambda i, b: (b[0], 0))],
            out_specs=pl.BlockSpec((1, 1), lambda i, b: (0, 0)),
        ),
        out_shape=jax.ShapeDtypeStruct((1, 1), jnp.int32),
    )(bstep.reshape((1,)), m_test, keys)
    m_star = jax.lax.dynamic_slice(keys, (star_idx[0, 0], 0), (1, D))
    msmt = jnp.concatenate([m_star, m_test], axis=0)
    score = pl.pallas_call(
        _stage2_body,
        grid=(NK2,),
        in_specs=[pl.BlockSpec((2, D), lambda j: (0, 0)),
                  pl.BlockSpec((BK2, D), lambda j: (j, 0)),
                  pl.BlockSpec((1, 1), lambda j: (0, 0))],
        out_specs=pl.BlockSpec((1, 1), lambda j: (0, 0)),
        out_shape=jax.ShapeDtypeStruct((1, 1), jnp.float32),
        scratch_shapes=[pltpu.SMEM((8,), jnp.float32)],
        compiler_params=pltpu.CompilerParams(
            dimension_semantics=("arbitrary",)),
    )(msmt, keys, s_star)

    return score[0, 0], min_d.reshape(32, 32)


# speedup vs baseline: 1.0221x; 1.0126x over previous
import jax
import jax.numpy as jnp
from jax.experimental import pallas as pl
from jax.experimental.pallas import tpu as pltpu

Q, K, D = 1024, 16384, 1536
BQ, BK = 256, 2048
NQ, NK = Q // BQ, K // BK
BK2 = 2048
NK2 = K // BK2
_INF = float("inf")
_EPS = 1e-12


def _stage1_body(q_ref, k_ref, mind_ref, sidx_ref, bstep_ref, sstar_ref,
                 fmin_ref, fstep_ref):
    j = pl.program_id(0)
    q = q_ref[...]
    NH = 4
    H = BK // NH
    ks = [k_ref[pl.ds(h * H, H), :] for h in range(NH)]
    kqs = [jax.lax.dot_general(kh, q, (((1,), (1,)), ((), ())),
                               preferred_element_type=jnp.float32)
           for kh in ks]
    k2s = [jnp.sum(kh * kh, axis=1, keepdims=True) for kh in ks]
    bmins = [jnp.min(k2 - 2.0 * kq, axis=0, keepdims=True)
             for k2, kq in zip(k2s, kqs)]
    bmin = bmins[0]
    for b in bmins[1:]:
        bmin = jnp.minimum(bmin, b)

    @pl.when(j == 0)
    def _():
        fmin_ref[...] = jnp.full((1, Q), _INF, jnp.float32)
        fstep_ref[...] = jnp.zeros((1, Q), jnp.int32)

    old_min = fmin_ref[...]
    old_step = fstep_ref[...]
    take = bmin < old_min
    new_min = jnp.where(take, bmin, old_min)
    new_step = jnp.where(take, j, old_step)
    fmin_ref[...] = new_min
    fstep_ref[...] = new_step

    @pl.when(j == NK - 1)
    def _():
        q2 = jax.lax.dot_general(jnp.ones((1, D), jnp.float32), q * q,
                                 (((1,), (1,)), ((), ())),
                                 precision=jax.lax.Precision.HIGHEST,
                                 preferred_element_type=jnp.float32)
        d2min = q2 + new_min
        mind_ref[...] = jnp.sqrt(jnp.maximum(d2min, _EPS))
        s_val = jnp.max(d2min)
        qio = jax.lax.broadcasted_iota(jnp.int32, (1, Q), 1)
        s_idx = jnp.min(jnp.where(d2min == s_val, qio, Q))
        bstep = jnp.sum(jnp.where(qio == s_idx, new_step, 0))
        sidx_ref[...] = jnp.full((1, 1), s_idx, jnp.int32)
        bstep_ref[...] = jnp.full((1, 1), bstep, jnp.int32)
        sstar_ref[...] = jnp.full((1, 1), jnp.sqrt(jnp.maximum(s_val, _EPS)),
                                  jnp.float32)


def _star_body(bstep_pref, mt_ref, k_ref, star_ref):
    k = k_ref[...]
    mt = mt_ref[...]
    kq = jax.lax.dot_general(k, mt, (((1,), (1,)), ((), ())),
                             preferred_element_type=jnp.float32)
    k2 = jnp.sum(k * k, axis=1, keepdims=True)
    e = k2 - 2.0 * kq
    v = jnp.min(e)
    io = jax.lax.broadcasted_iota(jnp.int32, (BK, 1), 0)
    loc = jnp.min(jnp.where(e == v, io, BK))
    star_ref[...] = jnp.full((1, 1), loc + bstep_pref[0] * BK, jnp.int32)


def _stage2_body(msmt_ref, k_ref, sstar_ref, out_ref, cand_ref):
    j = pl.program_id(0)
    k = k_ref[...]
    msmt = msmt_ref[...]
    kq = jax.lax.dot_general(k, msmt, (((1,), (1,)), ((), ())),
                             preferred_element_type=jnp.float32)
    m2 = jnp.sum(msmt * msmt, axis=1, keepdims=True)
    k2 = jnp.sum(k * k, axis=1, keepdims=True)
    ds = jnp.maximum(k2 - 2.0 * kq[:, 0:1] + m2[0:1, :], _EPS)
    dt = jnp.maximum(k2 - 2.0 * kq[:, 1:2] + m2[1:2, :], _EPS)
    io = jax.lax.broadcasted_iota(jnp.int32, (BK2, 1), 0)

    def top1(dvec):
        v = jnp.min(dvec)
        i1 = jnp.min(jnp.where(dvec == v, io, BK2))
        pay = jnp.sum(jnp.where(io == i1, dt, 0.0))
        return v, pay, i1

    bv1, bd1, i1 = top1(ds)
    ds_b = jnp.where(io == i1, _INF, ds)
    bv2, bd2, i2 = top1(ds_b)
    ds_c = jnp.where(io == i2, _INF, ds_b)
    bv3, bd3, _ = top1(ds_c)

    @pl.when(j == 0)
    def _():
        cand_ref[0] = _INF
        cand_ref[1] = _INF
        cand_ref[2] = _INF
        cand_ref[3] = 0.0
        cand_ref[4] = 0.0
        cand_ref[5] = 0.0

    rv1, rv2, rv3 = cand_ref[0], cand_ref[1], cand_ref[2]
    rd1, rd2, rd3 = cand_ref[3], cand_ref[4], cand_ref[5]

    c1 = bv1 < rv1
    o1v = jnp.where(c1, bv1, rv1)
    o1d = jnp.where(c1, bd1, rd1)
    a2 = bv1 < rv2
    A2v = jnp.where(a2, bv1, rv2)
    A2d = jnp.where(a2, bd1, rd2)
    A3v = jnp.where(a2, jnp.where(bv2 < rv2, bv2, rv2),
                    jnp.where(bv1 < rv3, bv1, rv3))
    A3d = jnp.where(a2, jnp.where(bv2 < rv2, bd2, rd2),
                    jnp.where(bv1 < rv3, bd1, rd3))
    b2c = bv2 < rv1
    B2v = jnp.where(b2c, bv2, rv1)
    B2d = jnp.where(b2c, bd2, rd1)
    B3v = jnp.where(b2c, jnp.where(bv3 < rv1, bv3, rv1),
                    jnp.where(bv2 < rv2, bv2, rv2))
    B3d = jnp.where(b2c, jnp.where(bv3 < rv1, bd3, rd1),
                    jnp.where(bv2 < rv2, bd2, rd2))
    o2v = jnp.where(c1, B2v, A2v)
    o2d = jnp.where(c1, B2d, A2d)
    o3v = jnp.where(c1, B3v, A3v)
    o3d = jnp.where(c1, B3d, A3d)
    cand_ref[0] = o1v
    cand_ref[1] = o2v
    cand_ref[2] = o3v
    cand_ref[3] = o1d
    cand_ref[4] = o2d
    cand_ref[5] = o3d

    @pl.when(j == NK2 - 1)
    def _():
        dc = jnp.sqrt(jnp.float32(D))
        s_star = sstar_ref[...]
        den = jnp.exp(jnp.sqrt(jnp.full((1, 1), o2d)) / dc) + \
            jnp.exp(jnp.sqrt(jnp.full((1, 1), o3d)) / dc)
        out_ref[...] = (1.0 - jnp.exp(s_star / dc) / den) * s_star


def kernel(queries, keys):
    min_d, s_idx, bstep, s_star = pl.pallas_call(
        _stage1_body,
        grid=(NK,),
        in_specs=[pl.BlockSpec((Q, D), lambda j: (0, 0)),
                  pl.BlockSpec((BK, D), lambda j: (j, 0))],
        out_specs=[pl.BlockSpec((1, Q), lambda j: (0, 0)),
                   pl.BlockSpec((1, 1), lambda j: (0, 0)),
                   pl.BlockSpec((1, 1), lambda j: (0, 0)),
                   pl.BlockSpec((1, 1), lambda j: (0, 0))],
        out_shape=[jax.ShapeDtypeStruct((1, Q), jnp.float32),
                   jax.ShapeDtypeStruct((1, 1), jnp.int32),
                   jax.ShapeDtypeStruct((1, 1), jnp.int32),
                   jax.ShapeDtypeStruct((1, 1), jnp.float32)],
        scratch_shapes=[pltpu.VMEM((1, Q), jnp.float32),
                        pltpu.VMEM((1, Q), jnp.int32)],
        compiler_params=pltpu.CompilerParams(
            dimension_semantics=("arbitrary",)),
    )(queries, keys)

    m_test = jax.lax.dynamic_slice(queries, (s_idx[0, 0], 0), (1, D))
    star_idx = pl.pallas_call(
        _star_body,
        grid_spec=pltpu.PrefetchScalarGridSpec(
            num_scalar_prefetch=1,
            grid=(1,),
            in_specs=[pl.BlockSpec((1, D), lambda i, b: (0, 0)),
                      pl.BlockSpec((BK, D), lambda i, b: (b[0], 0))],
            out_specs=pl.BlockSpec((1, 1), lambda i, b: (0, 0)),
        ),
        out_shape=jax.ShapeDtypeStruct((1, 1), jnp.int32),
    )(bstep.reshape((1,)), m_test, keys)
    m_star = jax.lax.dynamic_slice(keys, (star_idx[0, 0], 0), (1, D))
    msmt = jnp.concatenate([m_star, m_test], axis=0)
    score = pl.pallas_call(
        _stage2_body,
        grid=(NK2,),
        in_specs=[pl.BlockSpec((2, D), lambda j: (0, 0)),
                  pl.BlockSpec((BK2, D), lambda j: (j, 0)),
                  pl.BlockSpec((1, 1), lambda j: (0, 0))],
        out_specs=pl.BlockSpec((1, 1), lambda j: (0, 0)),
        out_shape=jax.ShapeDtypeStruct((1, 1), jnp.float32),
        scratch_shapes=[pltpu.SMEM((8,), jnp.float32)],
        compiler_params=pltpu.CompilerParams(
            dimension_semantics=("arbitrary",)),
    )(msmt, keys, s_star)

    return score[0, 0], min_d.reshape(32, 32)


# speedup vs baseline: 1.0642x; 1.0412x over previous
import jax
import jax.numpy as jnp
from jax.experimental import pallas as pl
from jax.experimental.pallas import tpu as pltpu

Q, K, D = 1024, 16384, 1536
BQ, BK = 256, 2048
NQ, NK = Q // BQ, K // BK
BK2 = 2048
NK2 = K // BK2
_INF = float("inf")
_EPS = 1e-12


def _stage1_body(q_ref, k_ref, mind_ref, sidx_ref, bstep_ref, sstar_ref,
                 fmin_ref, fstep_ref):
    j = pl.program_id(0)
    q = q_ref[...]
    NH = 4
    H = BK // NH
    ks = [k_ref[pl.ds(h * H, H), :] for h in range(NH)]
    kqs = [jax.lax.dot_general(kh, q, (((1,), (1,)), ((), ())),
                               preferred_element_type=jnp.float32)
           for kh in ks]
    k2s = [jnp.sum(kh * kh, axis=1, keepdims=True) for kh in ks]
    bmins = [jnp.min(k2 - 2.0 * kq, axis=0, keepdims=True)
             for k2, kq in zip(k2s, kqs)]
    bmin = bmins[0]
    for b in bmins[1:]:
        bmin = jnp.minimum(bmin, b)

    @pl.when(j == 0)
    def _():
        fmin_ref[...] = jnp.full((1, Q), _INF, jnp.float32)
        fstep_ref[...] = jnp.zeros((1, Q), jnp.int32)

    old_min = fmin_ref[...]
    old_step = fstep_ref[...]
    take = bmin < old_min
    new_min = jnp.where(take, bmin, old_min)
    new_step = jnp.where(take, j, old_step)
    fmin_ref[...] = new_min
    fstep_ref[...] = new_step

    @pl.when(j == NK - 1)
    def _():
        q2 = jax.lax.dot_general(jnp.ones((1, D), jnp.float32), q * q,
                                 (((1,), (1,)), ((), ())),
                                 precision=jax.lax.Precision.HIGHEST,
                                 preferred_element_type=jnp.float32)
        d2min = q2 + new_min
        mind_ref[...] = jnp.sqrt(jnp.maximum(d2min, _EPS))
        s_val = jnp.max(d2min)
        qio = jax.lax.broadcasted_iota(jnp.int32, (1, Q), 1)
        s_idx = jnp.min(jnp.where(d2min == s_val, qio, Q))
        bstep = jnp.sum(jnp.where(qio == s_idx, new_step, 0))
        sidx_ref[...] = jnp.full((1, 1), s_idx, jnp.int32)
        bstep_ref[...] = jnp.full((1, 1), bstep, jnp.int32)
        sstar_ref[...] = jnp.full((1, 1), jnp.sqrt(jnp.maximum(s_val, _EPS)),
                                  jnp.float32)


def _star_body(bstep_pref, mt_ref, k_ref, star_ref):
    k = k_ref[...]
    mt = mt_ref[...]
    kq = jax.lax.dot_general(k, mt, (((1,), (1,)), ((), ())),
                             preferred_element_type=jnp.float32)
    k2 = jnp.sum(k * k, axis=1, keepdims=True)
    e = k2 - 2.0 * kq
    v = jnp.min(e)
    io = jax.lax.broadcasted_iota(jnp.int32, (BK, 1), 0)
    loc = jnp.min(jnp.where(e == v, io, BK))
    star_ref[...] = jnp.full((1, 1), loc + bstep_pref[0] * BK, jnp.int32)


def _stage2_body(msmt_ref, k_ref, sstar_ref, out_ref, cand_ref, dt_ref):
    j = pl.program_id(0)
    k = k_ref[...]
    msmt = msmt_ref[...]
    kq = jax.lax.dot_general(k, msmt, (((1,), (1,)), ((), ())),
                             preferred_element_type=jnp.float32)
    m2 = jnp.sum(msmt * msmt, axis=1, keepdims=True)
    k2 = jnp.sum(k * k, axis=1, keepdims=True)
    ds = jnp.maximum(k2 - 2.0 * kq[:, 0:1] + m2[0:1, :], _EPS)
    dt = jnp.maximum(k2 - 2.0 * kq[:, 1:2] + m2[1:2, :], _EPS)
    io = jax.lax.broadcasted_iota(jnp.int32, (BK2, 1), 0)

    dt_ref[...] = dt

    def top1(dvec):
        v = jnp.min(dvec)
        i1 = jnp.min(jnp.where(dvec == v, io, BK2))
        pay = dt_ref[i1, 0]
        return v, pay, i1

    bv1, bd1, i1 = top1(ds)
    ds_b = jnp.where(io == i1, _INF, ds)
    bv2, bd2, i2 = top1(ds_b)
    ds_c = jnp.where(io == i2, _INF, ds_b)
    bv3, bd3, _ = top1(ds_c)

    @pl.when(j == 0)
    def _():
        cand_ref[0] = _INF
        cand_ref[1] = _INF
        cand_ref[2] = _INF
        cand_ref[3] = 0.0
        cand_ref[4] = 0.0
        cand_ref[5] = 0.0

    rv1, rv2, rv3 = cand_ref[0], cand_ref[1], cand_ref[2]
    rd1, rd2, rd3 = cand_ref[3], cand_ref[4], cand_ref[5]

    c1 = bv1 < rv1
    o1v = jnp.where(c1, bv1, rv1)
    o1d = jnp.where(c1, bd1, rd1)
    a2 = bv1 < rv2
    A2v = jnp.where(a2, bv1, rv2)
    A2d = jnp.where(a2, bd1, rd2)
    A3v = jnp.where(a2, jnp.where(bv2 < rv2, bv2, rv2),
                    jnp.where(bv1 < rv3, bv1, rv3))
    A3d = jnp.where(a2, jnp.where(bv2 < rv2, bd2, rd2),
                    jnp.where(bv1 < rv3, bd1, rd3))
    b2c = bv2 < rv1
    B2v = jnp.where(b2c, bv2, rv1)
    B2d = jnp.where(b2c, bd2, rd1)
    B3v = jnp.where(b2c, jnp.where(bv3 < rv1, bv3, rv1),
                    jnp.where(bv2 < rv2, bv2, rv2))
    B3d = jnp.where(b2c, jnp.where(bv3 < rv1, bd3, rd1),
                    jnp.where(bv2 < rv2, bd2, rd2))
    o2v = jnp.where(c1, B2v, A2v)
    o2d = jnp.where(c1, B2d, A2d)
    o3v = jnp.where(c1, B3v, A3v)
    o3d = jnp.where(c1, B3d, A3d)
    cand_ref[0] = o1v
    cand_ref[1] = o2v
    cand_ref[2] = o3v
    cand_ref[3] = o1d
    cand_ref[4] = o2d
    cand_ref[5] = o3d

    @pl.when(j == NK2 - 1)
    def _():
        dc = jnp.sqrt(jnp.float32(D))
        s_star = sstar_ref[...]
        den = jnp.exp(jnp.sqrt(jnp.full((1, 1), o2d)) / dc) + \
            jnp.exp(jnp.sqrt(jnp.full((1, 1), o3d)) / dc)
        out_ref[...] = (1.0 - jnp.exp(s_star / dc) / den) * s_star


def kernel(queries, keys):
    min_d, s_idx, bstep, s_star = pl.pallas_call(
        _stage1_body,
        grid=(NK,),
        in_specs=[pl.BlockSpec((Q, D), lambda j: (0, 0)),
                  pl.BlockSpec((BK, D), lambda j: (j, 0))],
        out_specs=[pl.BlockSpec((1, Q), lambda j: (0, 0)),
                   pl.BlockSpec((1, 1), lambda j: (0, 0)),
                   pl.BlockSpec((1, 1), lambda j: (0, 0)),
                   pl.BlockSpec((1, 1), lambda j: (0, 0))],
        out_shape=[jax.ShapeDtypeStruct((1, Q), jnp.float32),
                   jax.ShapeDtypeStruct((1, 1), jnp.int32),
                   jax.ShapeDtypeStruct((1, 1), jnp.int32),
                   jax.ShapeDtypeStruct((1, 1), jnp.float32)],
        scratch_shapes=[pltpu.VMEM((1, Q), jnp.float32),
                        pltpu.VMEM((1, Q), jnp.int32)],
        compiler_params=pltpu.CompilerParams(
            dimension_semantics=("arbitrary",)),
    )(queries, keys)

    m_test = jax.lax.dynamic_slice(queries, (s_idx[0, 0], 0), (1, D))
    star_idx = pl.pallas_call(
        _star_body,
        grid_spec=pltpu.PrefetchScalarGridSpec(
            num_scalar_prefetch=1,
            grid=(1,),
            in_specs=[pl.BlockSpec((1, D), lambda i, b: (0, 0)),
                      pl.BlockSpec((BK, D), lambda i, b: (b[0], 0))],
            out_specs=pl.BlockSpec((1, 1), lambda i, b: (0, 0)),
        ),
        out_shape=jax.ShapeDtypeStruct((1, 1), jnp.int32),
    )(bstep.reshape((1,)), m_test, keys)
    m_star = jax.lax.dynamic_slice(keys, (star_idx[0, 0], 0), (1, D))
    msmt = jnp.concatenate([m_star, m_test], axis=0)
    score = pl.pallas_call(
        _stage2_body,
        grid=(NK2,),
        in_specs=[pl.BlockSpec((2, D), lambda j: (0, 0)),
                  pl.BlockSpec((BK2, D), lambda j: (j, 0)),
                  pl.BlockSpec((1, 1), lambda j: (0, 0))],
        out_specs=pl.BlockSpec((1, 1), lambda j: (0, 0)),
        out_shape=jax.ShapeDtypeStruct((1, 1), jnp.float32),
        scratch_shapes=[pltpu.SMEM((8,), jnp.float32),
                        pltpu.VMEM((BK2, 1), jnp.float32)],
        compiler_params=pltpu.CompilerParams(
            dimension_semantics=("arbitrary",)),
    )(msmt, keys, s_star)

    return score[0, 0], min_d.reshape(32, 32)
